# baseline (speedup 1.0000x reference)
import functools

import jax
import jax.numpy as jnp
from jax import lax
from jax.experimental import pallas as pl
from jax.experimental.pallas import tpu as pltpu
from jax.experimental.pallas import tpu_sc as plsc

HID = 128
K = 512


def _conv(x, w, b, stride, pad):
    y = lax.conv_general_dilated(
        x, w, (stride, stride), ((pad, pad), (pad, pad)),
        dimension_numbers=('NCHW', 'OIHW', 'NCHW'))
    return y + b[None, :, None, None]


def _conv_t(x, w, b, stride, pad):
    kh, kw = w.shape[2], w.shape[3]
    w_f = jnp.transpose(jnp.flip(w, (2, 3)), (1, 0, 2, 3))
    y = lax.conv_general_dilated(
        x, w_f, (1, 1), ((kh - 1 - pad, kh - 1 - pad), (kw - 1 - pad, kw - 1 - pad)),
        lhs_dilation=(stride, stride), dimension_numbers=('NCHW', 'OIHW', 'NCHW'))
    return y + b[None, :, None, None]


def _resblock(x, w1, b1, w2, b2):
    h = _conv(jax.nn.relu(x), w1, b1, 1, 1)
    h = _conv(jax.nn.relu(h), w2, b2, 1, 0)
    return x + h


_TILE_S = 3136


def _vq_argmin_body(ze_ref, ze2_ref, emb_ref, idx_ref):
    ze = ze_ref[0]
    emb = emb_ref[...]
    cross = jnp.dot(emb, ze)
    e2 = jnp.sum(emb * emb, axis=1)
    dist = (ze2_ref[0] + e2[:, None]) - 2.0 * cross
    minv = jnp.min(dist, axis=0, keepdims=True)
    kio = lax.broadcasted_iota(jnp.int32, (K, _TILE_S), 0)
    idx_ref[0, 0, :] = jnp.min(jnp.where(dist == minv, kio, K), axis=0)


def _vq_argmin(ze_mat, ze2_mat, emb):
    return pl.pallas_call(
        _vq_argmin_body,
        grid=(2,),
        in_specs=[
            pl.BlockSpec((1, HID, _TILE_S), lambda b: (b, 0, 0)),
            pl.BlockSpec((1, 1, _TILE_S), lambda b: (b, 0, 0)),
            pl.BlockSpec((K, HID), lambda b: (0, 0)),
        ],
        out_specs=pl.BlockSpec((1, 1, _TILE_S), lambda b: (b, 0, 0)),
        out_shape=jax.ShapeDtypeStruct((2, 1, 3136), jnp.int32),
    )(ze_mat, ze2_mat, emb)


_B_PAD = 6400
_NW = 32
_B_PER_W = _B_PAD // _NW


def _make_sc_gather():
    mesh = plsc.VectorSubcoreMesh(core_axis_name="c", subcore_axis_name="s")

    @functools.partial(
        pl.kernel, mesh=mesh,
        out_type=jax.ShapeDtypeStruct((_B_PAD, HID), jnp.float32),
        scratch_types=[
            pltpu.VMEM((_B_PER_W,), jnp.int32),
            pltpu.VMEM((_B_PER_W, HID), jnp.float32),
            pltpu.SemaphoreType.DMA,
        ],
    )
    def gather_k(table_hbm, idx_hbm, out_hbm, idx_v, rows_v, sem):
        wid = lax.axis_index("s") * 2 + lax.axis_index("c")
        base = wid * _B_PER_W
        pltpu.sync_copy(idx_hbm.at[pl.ds(base, _B_PER_W)], idx_v)
        pltpu.async_copy(table_hbm.at[idx_v], rows_v, sem).wait()
        pltpu.sync_copy(rows_v, out_hbm.at[pl.ds(base, _B_PER_W)])

    return gather_k


_sc_gather_cache = []


def _sc_gather(table, idx):
    if not _sc_gather_cache:
        _sc_gather_cache.append(_make_sc_gather())
    return _sc_gather_cache[0](table, idx)


def kernel(x, enc_w0, enc_b0, enc_w1, enc_b1, enc_w2, enc_b2, erb1_w1, erb1_b1, erb1_w2, erb1_b2, erb2_w1, erb2_b1, erb2_w2, erb2_b2, emb, dec_w0, dec_b0, drb1_w1, drb1_b1, drb1_w2, drb1_b2, drb2_w1, drb2_b1, drb2_w2, drb2_b2, dct_w1, dct_b1, dct_w2, dct_b2):
    ze = jax.nn.relu(_conv(x, enc_w0, enc_b0, 2, 1))
    ze = jax.nn.relu(_conv(ze, enc_w1, enc_b1, 2, 1))
    ze = _conv(ze, enc_w2, enc_b2, 1, 1)
    ze = _resblock(ze, erb1_w1, erb1_b1, erb1_w2, erb1_b2)
    ze = _resblock(ze, erb2_w1, erb2_b1, erb2_w2, erb2_b2)

    ze2 = jnp.sum(ze * ze, axis=1)
    ze_mat = ze.reshape(2, HID, 3136)
    ze2_mat = ze2.reshape(2, 1, 3136)
    idx = _vq_argmin(ze_mat, ze2_mat, emb)
    idx_flat = idx.reshape(6272)
    idx_pad = jnp.concatenate(
        [idx_flat, jnp.zeros((_B_PAD - 6272,), jnp.int32)])
    zq_rows = _sc_gather(emb, idx_pad)
    zq = jnp.transpose(
        zq_rows[:6272].reshape(2, 3136, HID), (0, 2, 1)).reshape(2, HID, 56, 56)

    dec_in = ze + lax.stop_gradient(zq - ze)
    h = _conv(dec_in, dec_w0, dec_b0, 1, 1)
    h = _resblock(h, drb1_w1, drb1_b1, drb1_w2, drb1_b2)
    h = _resblock(h, drb2_w1, drb2_b1, drb2_w2, drb2_b2)
    h = jax.nn.relu(_conv_t(h, dct_w1, dct_b1, 2, 1))
    x_hat = _conv_t(h, dct_w2, dct_b2, 2, 1)
    return (x_hat, ze, zq)


# speedup vs baseline: 1.2660x; 1.2660x over previous
import functools

import jax
import jax.numpy as jnp
from jax import lax
from jax.experimental import pallas as pl
from jax.experimental.pallas import tpu as pltpu
from jax.experimental.pallas import tpu_sc as plsc

HID = 128
K = 512


def _conv(x, w, b, stride, pad):
    y = lax.conv_general_dilated(
        x, w, (stride, stride), ((pad, pad), (pad, pad)),
        dimension_numbers=('NCHW', 'OIHW', 'NCHW'))
    return y + b[None, :, None, None]


def _conv_t(x, w, b, stride, pad):
    kh, kw = w.shape[2], w.shape[3]
    w_f = jnp.transpose(jnp.flip(w, (2, 3)), (1, 0, 2, 3))
    y = lax.conv_general_dilated(
        x, w_f, (1, 1), ((kh - 1 - pad, kh - 1 - pad), (kw - 1 - pad, kw - 1 - pad)),
        lhs_dilation=(stride, stride), dimension_numbers=('NCHW', 'OIHW', 'NCHW'))
    return y + b[None, :, None, None]


def _resblock(x, w1, b1, w2, b2):
    h = _conv(jax.nn.relu(x), w1, b1, 1, 1)
    h = _conv(jax.nn.relu(h), w2, b2, 1, 0)
    return x + h


_TILE_S = 3136


def _vq_body(ze_ref, ze2_ref, emb_ref, zq_ref):
    ze = ze_ref[0]
    emb = emb_ref[...]
    cross = jnp.dot(emb, ze)
    e2 = jnp.sum(emb * emb, axis=1)
    dist = (ze2_ref[0] + e2[:, None]) - 2.0 * cross
    minv = jnp.min(dist, axis=0, keepdims=True)
    kio = lax.broadcasted_iota(jnp.int32, (K, _TILE_S), 0)
    idx = jnp.min(jnp.where(dist == minv, kio, K), axis=0)
    onehot = (kio == idx[None, :]).astype(jnp.float32)
    zq_ref[0] = lax.dot_general(emb, onehot, (((0,), (0,)), ((), ())))


def _vq_lookup(ze_mat, ze2_mat, emb):
    return pl.pallas_call(
        _vq_body,
        grid=(2,),
        in_specs=[
            pl.BlockSpec((1, HID, _TILE_S), lambda b: (b, 0, 0)),
            pl.BlockSpec((1, 1, _TILE_S), lambda b: (b, 0, 0)),
            pl.BlockSpec((K, HID), lambda b: (0, 0)),
        ],
        out_specs=pl.BlockSpec((1, HID, _TILE_S), lambda b: (b, 0, 0)),
        out_shape=jax.ShapeDtypeStruct((2, HID, 3136), jnp.float32),
    )(ze_mat, ze2_mat, emb)


_B_PAD = 6400
_NW = 32
_B_PER_W = _B_PAD // _NW


def _make_sc_gather():
    mesh = plsc.VectorSubcoreMesh(core_axis_name="c", subcore_axis_name="s")

    @functools.partial(
        pl.kernel, mesh=mesh,
        out_type=jax.ShapeDtypeStruct((_B_PAD, HID), jnp.float32),
        scratch_types=[
            pltpu.VMEM((_B_PER_W,), jnp.int32),
            pltpu.VMEM((_B_PER_W, HID), jnp.float32),
            pltpu.SemaphoreType.DMA,
        ],
    )
    def gather_k(table_hbm, idx_hbm, out_hbm, idx_v, rows_v, sem):
        wid = lax.axis_index("s") * 2 + lax.axis_index("c")
        base = wid * _B_PER_W
        pltpu.sync_copy(idx_hbm.at[pl.ds(base, _B_PER_W)], idx_v)
        pltpu.async_copy(table_hbm.at[idx_v], rows_v, sem).wait()
        pltpu.sync_copy(rows_v, out_hbm.at[pl.ds(base, _B_PER_W)])

    return gather_k


_sc_gather_cache = []


def _sc_gather(table, idx):
    if not _sc_gather_cache:
        _sc_gather_cache.append(_make_sc_gather())
    return _sc_gather_cache[0](table, idx)


def kernel(x, enc_w0, enc_b0, enc_w1, enc_b1, enc_w2, enc_b2, erb1_w1, erb1_b1, erb1_w2, erb1_b2, erb2_w1, erb2_b1, erb2_w2, erb2_b2, emb, dec_w0, dec_b0, drb1_w1, drb1_b1, drb1_w2, drb1_b2, drb2_w1, drb2_b1, drb2_w2, drb2_b2, dct_w1, dct_b1, dct_w2, dct_b2):
    ze = jax.nn.relu(_conv(x, enc_w0, enc_b0, 2, 1))
    ze = jax.nn.relu(_conv(ze, enc_w1, enc_b1, 2, 1))
    ze = _conv(ze, enc_w2, enc_b2, 1, 1)
    ze = _resblock(ze, erb1_w1, erb1_b1, erb1_w2, erb1_b2)
    ze = _resblock(ze, erb2_w1, erb2_b1, erb2_w2, erb2_b2)

    ze2 = jnp.sum(ze * ze, axis=1)
    ze_mat = ze.reshape(2, HID, 3136)
    ze2_mat = ze2.reshape(2, 1, 3136)
    zq = _vq_lookup(ze_mat, ze2_mat, emb).reshape(2, HID, 56, 56)

    dec_in = ze + lax.stop_gradient(zq - ze)
    h = _conv(dec_in, dec_w0, dec_b0, 1, 1)
    h = _resblock(h, drb1_w1, drb1_b1, drb1_w2, drb1_b2)
    h = _resblock(h, drb2_w1, drb2_b1, drb2_w2, drb2_b2)
    h = jax.nn.relu(_conv_t(h, dct_w1, dct_b1, 2, 1))
    x_hat = _conv_t(h, dct_w2, dct_b2, 2, 1)
    return (x_hat, ze, zq)


# speedup vs baseline: 1.2660x; 1.0000x over previous
import jax
import jax.numpy as jnp
from jax import lax
from jax.experimental import pallas as pl

HID = 128
K = 512


def _conv(x, w, b, stride, pad):
    y = lax.conv_general_dilated(
        x, w, (stride, stride), ((pad, pad), (pad, pad)),
        dimension_numbers=('NCHW', 'OIHW', 'NCHW'))
    return y + b[None, :, None, None]


def _conv_t(x, w, b, stride, pad):
    kh, kw = w.shape[2], w.shape[3]
    w_f = jnp.transpose(jnp.flip(w, (2, 3)), (1, 0, 2, 3))
    y = lax.conv_general_dilated(
        x, w_f, (1, 1), ((kh - 1 - pad, kh - 1 - pad), (kw - 1 - pad, kw - 1 - pad)),
        lhs_dilation=(stride, stride), dimension_numbers=('NCHW', 'OIHW', 'NCHW'))
    return y + b[None, :, None, None]


def _resblock(x, w1, b1, w2, b2):
    h = _conv(jax.nn.relu(x), w1, b1, 1, 1)
    h = _conv(jax.nn.relu(h), w2, b2, 1, 0)
    return x + h


_TILE_S = 3136


def _vq_body(ze_ref, ze2_ref, emb_ref, zq_ref):
    ze = ze_ref[0]
    emb = emb_ref[...]
    cross = jnp.dot(emb, ze)
    e2 = jnp.sum(emb * emb, axis=1)
    dist = (ze2_ref[0] + e2[:, None]) - 2.0 * cross
    minv = jnp.min(dist, axis=0, keepdims=True)
    kio = lax.broadcasted_iota(jnp.int32, (K, _TILE_S), 0)
    idx = jnp.min(jnp.where(dist == minv, kio, K), axis=0)
    onehot = (kio == idx[None, :]).astype(jnp.float32)
    zq_ref[0] = lax.dot_general(emb, onehot, (((0,), (0,)), ((), ())))


def _vq_lookup(ze_mat, ze2_mat, emb):
    return pl.pallas_call(
        _vq_body,
        grid=(2,),
        in_specs=[
            pl.BlockSpec((1, HID, _TILE_S), lambda b: (b, 0, 0)),
            pl.BlockSpec((1, 1, _TILE_S), lambda b: (b, 0, 0)),
            pl.BlockSpec((K, HID), lambda b: (0, 0)),
        ],
        out_specs=pl.BlockSpec((1, HID, _TILE_S), lambda b: (b, 0, 0)),
        out_shape=jax.ShapeDtypeStruct((2, HID, 3136), jnp.float32),
    )(ze_mat, ze2_mat, emb)


def kernel(x, enc_w0, enc_b0, enc_w1, enc_b1, enc_w2, enc_b2, erb1_w1, erb1_b1, erb1_w2, erb1_b2, erb2_w1, erb2_b1, erb2_w2, erb2_b2, emb, dec_w0, dec_b0, drb1_w1, drb1_b1, drb1_w2, drb1_b2, drb2_w1, drb2_b1, drb2_w2, drb2_b2, dct_w1, dct_b1, dct_w2, dct_b2):
    ze = jax.nn.relu(_conv(x, enc_w0, enc_b0, 2, 1))
    ze = jax.nn.relu(_conv(ze, enc_w1, enc_b1, 2, 1))
    ze = _conv(ze, enc_w2, enc_b2, 1, 1)
    ze = _resblock(ze, erb1_w1, erb1_b1, erb1_w2, erb1_b2)
    ze = _resblock(ze, erb2_w1, erb2_b1, erb2_w2, erb2_b2)

    ze2 = jnp.sum(ze * ze, axis=1)
    ze_mat = ze.reshape(2, HID, 3136)
    ze2_mat = ze2.reshape(2, 1, 3136)
    zq = _vq_lookup(ze_mat, ze2_mat, emb).reshape(2, HID, 56, 56)

    dec_in = ze + lax.stop_gradient(zq - ze)
    h = _conv(dec_in, dec_w0, dec_b0, 1, 1)
    h = _resblock(h, drb1_w1, drb1_b1, drb1_w2, drb1_b2)
    h = _resblock(h, drb2_w1, drb2_b1, drb2_w2, drb2_b2)
    h = jax.nn.relu(_conv_t(h, dct_w1, dct_b1, 2, 1))
    x_hat = _conv_t(h, dct_w2, dct_b2, 2, 1)
    return (x_hat, ze, zq)
